# Initial kernel scaffold; baseline (speedup 1.0000x reference)
import jax
import jax.numpy as jnp
from jax.experimental import pallas as pl


def kernel(x, Wqk_w, Wqk_b, Wv_w, Wv_b, unify_w, unify_b, n1_g, n1_b, ff_w1, ff_b1, ff_w2, ff_b2, n2_g, n2_b, hashM):
    raise NotImplementedError("write your pallas kernel here")


# speedup vs baseline: 1.4144x; 1.4144x over previous
import functools

import jax
import jax.numpy as jnp
from jax import lax
from jax.experimental import pallas as pl
from jax.experimental.pallas import tpu as pltpu
from jax.experimental.pallas import tpu_sc as plsc

DMODEL = 1024
H = DMODEL // 2
DQK = 64
HEADS = 16
HD = HEADS * DQK
FF = 2048
NB = 64
PEN = 100000.0
D = 2
N = 4096
T = D * N
CH = 128
NCHUNK = T // CH

_F32 = jnp.float32


def _hash_body(x_ref, wqk_ref, bqk_ref, hm_ref, x2_ref, h_ref):
    x2 = x_ref[:, H:]
    x2_ref[...] = x2
    qk = jnp.dot(x2, wqk_ref[...], preferred_element_type=_F32) + bqk_ref[...]
    proj = jnp.dot(qk, hm_ref[...], preferred_element_type=_F32)
    blk = proj.shape[0]
    m = jnp.maximum(jnp.max(proj, axis=1, keepdims=True),
                    jnp.max(-proj, axis=1, keepdims=True))
    io = lax.broadcasted_iota(jnp.int32, (blk, NB // 2), 1)
    big = jnp.int32(2 * NB)
    i1 = jnp.min(jnp.where(proj == m, io, big), axis=1, keepdims=True)
    i2 = jnp.min(jnp.where(-proj == m, io + (NB // 2), big), axis=1,
                 keepdims=True)
    h_ref[...] = jnp.minimum(i1, i2)


def _hashes_and_x2(xf, wqk, bqk, hm):
    blk = 512
    grid = (T // blk,)
    return pl.pallas_call(
        _hash_body,
        grid=grid,
        in_specs=[
            pl.BlockSpec((blk, DMODEL), lambda i: (i, 0)),
            pl.BlockSpec((H, HD), lambda i: (0, 0)),
            pl.BlockSpec((1, HD), lambda i: (0, 0)),
            pl.BlockSpec((HD, NB // 2), lambda i: (0, 0)),
        ],
        out_specs=[
            pl.BlockSpec((blk, H), lambda i: (i, 0)),
            pl.BlockSpec((blk, 1), lambda i: (i, 0)),
        ],
        out_shape=[
            jax.ShapeDtypeStruct((T, H), _F32),
            jax.ShapeDtypeStruct((T, 1), jnp.int32),
        ],
    )(xf, wqk, bqk, hm)


def _perm_body(h_ref, g_ref):
    b = pl.program_id(0)
    h = h_ref[0]
    buckets = lax.broadcasted_iota(jnp.int32, (N, NB), 1)
    onehot = (h == buckets).astype(_F32)
    counts = jnp.sum(onehot, axis=0, keepdims=True)
    r0 = lax.broadcasted_iota(jnp.int32, (NB, NB), 0)
    c0 = lax.broadcasted_iota(jnp.int32, (NB, NB), 1)
    lt = (r0 < c0).astype(_F32)
    starts = jnp.dot(counts, lt, preferred_element_type=_F32)

    tb = 512
    nblk = N // tb
    r1 = lax.broadcasted_iota(jnp.int32, (tb, tb), 0)
    c1 = lax.broadcasted_iota(jnp.int32, (tb, tb), 1)
    lstrict = (c1 < r1).astype(_F32)

    run = starts
    pbs = []
    for i in range(nblk):
        ob = lax.slice(onehot, (i * tb, 0), ((i + 1) * tb, NB))
        cb = jnp.dot(lstrict, ob, preferred_element_type=_F32) + run
        pbs.append(jnp.sum(cb * ob, axis=1, keepdims=True))
        run = run + jnp.sum(ob, axis=0, keepdims=True)

    for kb in range(nblk):
        kio = lax.broadcasted_iota(jnp.int32, (tb, tb), 1).astype(_F32) \
            + _F32(kb * tb)
        acc = jnp.zeros((1, tb), _F32)
        for i in range(nblk):
            qb = (pbs[i] == kio).astype(_F32)
            tcol = lax.broadcasted_iota(jnp.int32, (tb, 1), 0).astype(_F32) \
                + _F32(i * tb)
            acc = acc + jnp.sum(qb * tcol, axis=0, keepdims=True)
        g_ref[0, kb, :] = acc[0].astype(jnp.int32) + b * N


def _sorted_gather_indices(hashes):
    h3 = hashes.reshape(D, N, 1)
    g = pl.pallas_call(
        _perm_body,
        grid=(D,),
        in_specs=[pl.BlockSpec((1, N, 1), lambda b: (b, 0, 0))],
        out_specs=pl.BlockSpec((1, N // 512, 512), lambda b: (b, 0, 0)),
        out_shape=jax.ShapeDtypeStruct((D, N // 512, 512), jnp.int32),
    )(h3)
    return g.reshape(T)


def _sc_gather(x2f, gidx):
    info = plsc.get_sparse_core_info()
    nw = info.num_cores * info.num_subcores
    per_w = T // nw
    rows_chunk = 32
    nchunk = per_w // rows_chunk
    mesh = plsc.VectorSubcoreMesh(core_axis_name="c", subcore_axis_name="s")

    @functools.partial(
        pl.kernel,
        mesh=mesh,
        out_type=jax.ShapeDtypeStruct((T, H), _F32),
        scratch_types=[
            pltpu.VMEM((rows_chunk,), jnp.int32),
            pltpu.VMEM((rows_chunk, H), _F32),
            pltpu.SemaphoreType.DMA,
        ],
    )
    def gather_k(tab_hbm, idx_hbm, out_hbm, idx_v, rows_v, sem):
        wid = lax.axis_index("s") * info.num_cores + lax.axis_index("c")
        base = wid * per_w
        for j in range(nchunk):
            off = base + j * rows_chunk
            pltpu.sync_copy(idx_hbm.at[pl.ds(off, rows_chunk)], idx_v)
            pltpu.async_copy(tab_hbm.at[idx_v], rows_v, sem).wait()
            pltpu.sync_copy(rows_v, out_hbm.at[pl.ds(off, rows_chunk)])

    return gather_k(x2f, gidx)


def _tail_body(x_ref, x2g_ref, wqk_ref, bqk_ref, wv_ref, bv_ref,
               wu_ref, bu_ref, n1g_ref, n1b_ref,
               w1_ref, b1_ref, w2_ref, b2_ref, n2g_ref, n2b_ref, o_ref):
    x2g = x2g_ref[...]
    qkf = jnp.dot(x2g, wqk_ref[...], preferred_element_type=_F32) + bqk_ref[...]
    valsf = jnp.dot(x2g, wv_ref[...], preferred_element_type=_F32) + bv_ref[...]
    s = lax.dot_general(qkf, qkf, (((1,), (1,)), ((), ())),
                        preferred_element_type=_F32) * _F32(1.0 / 8.0)
    rr = lax.broadcasted_iota(jnp.int32, (CH, CH), 0)
    cc = lax.broadcasted_iota(jnp.int32, (CH, CH), 1)
    s = jnp.where(rr == cc, s / _F32(PEN), s)
    attn = jnp.dot(s, valsf, preferred_element_type=_F32)
    ua = jnp.dot(attn, wu_ref[...], preferred_element_type=_F32) + bu_ref[...]

    x1 = x_ref[:, :H]
    x2 = x_ref[:, H:]
    t1 = x1 + ua
    mu = jnp.mean(t1, axis=1, keepdims=True)
    var = jnp.mean((t1 - mu) ** 2, axis=1, keepdims=True)
    y1 = (t1 - mu) / jnp.sqrt(var + _F32(1e-5)) * n1g_ref[...] + n1b_ref[...]

    ff = jnp.maximum(jnp.dot(y1, w1_ref[...], preferred_element_type=_F32)
                     + b1_ref[...], _F32(0.0))
    ffo = jnp.dot(ff, w2_ref[...], preferred_element_type=_F32) + b2_ref[...]
    t2 = x2 + ffo
    mu2 = jnp.mean(t2, axis=1, keepdims=True)
    var2 = jnp.mean((t2 - mu2) ** 2, axis=1, keepdims=True)
    y2 = (t2 - mu2) / jnp.sqrt(var2 + _F32(1e-5)) * n2g_ref[...] + n2b_ref[...]

    o_ref[:, :H] = y1
    o_ref[:, H:] = y2


def _attention_tail(xf, x2g, wqk, bqk, wv, bv, wu, bu, n1g, n1b,
                    w1, b1, w2, b2, n2g, n2b):
    res = lambda shape: pl.BlockSpec(shape, lambda c: tuple(0 for _ in shape))
    return pl.pallas_call(
        _tail_body,
        grid=(NCHUNK,),
        in_specs=[
            pl.BlockSpec((CH, DMODEL), lambda c: (c, 0)),
            pl.BlockSpec((CH, H), lambda c: (c, 0)),
            res((H, HD)), res((1, HD)),
            res((H, HD)), res((1, HD)),
            res((HD, H)), res((1, H)),
            res((1, H)), res((1, H)),
            res((H, FF)), res((1, FF)),
            res((FF, H)), res((1, H)),
            res((1, H)), res((1, H)),
        ],
        out_specs=pl.BlockSpec((CH, DMODEL), lambda c: (c, 0)),
        out_shape=jax.ShapeDtypeStruct((T, DMODEL), _F32),
    )(xf, x2g, wqk, bqk, wv, bv, wu, bu, n1g, n1b, w1, b1, w2, b2, n2g, n2b)


def kernel(x, Wqk_w, Wqk_b, Wv_w, Wv_b, unify_w, unify_b, n1_g, n1_b,
           ff_w1, ff_b1, ff_w2, ff_b2, n2_g, n2_b, hashM):
    xf = x.reshape(T, DMODEL)
    r = lambda v: v.reshape(1, -1)
    x2f, hashes = _hashes_and_x2(xf, Wqk_w, r(Wqk_b), hashM)
    gidx = _sorted_gather_indices(hashes)
    x2g = _sc_gather(x2f, gidx)
    out = _attention_tail(xf, x2g, Wqk_w, r(Wqk_b), Wv_w, r(Wv_b),
                          unify_w, r(unify_b), r(n1_g), r(n1_b),
                          ff_w1, r(ff_b1), ff_w2, r(ff_b2), r(n2_g), r(n2_b))
    return out.reshape(D, N, DMODEL)


# speedup vs baseline: 1.8372x; 1.2989x over previous
import functools

import jax
import jax.numpy as jnp
from jax import lax
from jax.experimental import pallas as pl
from jax.experimental.pallas import tpu as pltpu
from jax.experimental.pallas import tpu_sc as plsc

DMODEL = 1024
H = DMODEL // 2
DQK = 64
HEADS = 16
HD = HEADS * DQK
FF = 2048
NB = 64
PEN = 100000.0
D = 2
N = 4096
T = D * N
CH = 128
BC = 2
NCHUNK = T // CH

_F32 = jnp.float32


def _hash_body(x_ref, wqk_ref, bqk_ref, hm_ref, x2_ref, h_ref):
    x2 = x_ref[...]
    x2_ref[...] = x2
    qk = jnp.dot(x2, wqk_ref[...], preferred_element_type=_F32) + bqk_ref[...]
    proj = jnp.dot(qk, hm_ref[...], preferred_element_type=_F32)
    blk = proj.shape[0]
    m = jnp.maximum(jnp.max(proj, axis=1, keepdims=True),
                    jnp.max(-proj, axis=1, keepdims=True))
    io = lax.broadcasted_iota(jnp.int32, (blk, NB // 2), 1)
    big = jnp.int32(2 * NB)
    i1 = jnp.min(jnp.where(proj == m, io, big), axis=1, keepdims=True)
    i2 = jnp.min(jnp.where(-proj == m, io + (NB // 2), big), axis=1,
                 keepdims=True)
    h_ref[...] = jnp.minimum(i1, i2)


def _hashes_and_x2(xf, wqk, bqk, hm):
    blk = 512
    grid = (T // blk,)
    return pl.pallas_call(
        _hash_body,
        grid=grid,
        in_specs=[
            pl.BlockSpec((blk, H), lambda i: (i, 1)),
            pl.BlockSpec((H, HD), lambda i: (0, 0)),
            pl.BlockSpec((1, HD), lambda i: (0, 0)),
            pl.BlockSpec((HD, NB // 2), lambda i: (0, 0)),
        ],
        out_specs=[
            pl.BlockSpec((blk, H), lambda i: (i, 0)),
            pl.BlockSpec((blk, 1), lambda i: (i, 0)),
        ],
        out_shape=[
            jax.ShapeDtypeStruct((T, H), _F32),
            jax.ShapeDtypeStruct((T, 1), jnp.int32),
        ],
    )(xf, wqk, bqk, hm)


def _perm_body(h_ref, g_ref):
    b = pl.program_id(0)
    h = h_ref[0]
    buckets = lax.broadcasted_iota(jnp.int32, (N, NB), 1)
    onehot = (h == buckets).astype(_F32)
    counts = jnp.sum(onehot, axis=0, keepdims=True)
    r0 = lax.broadcasted_iota(jnp.int32, (NB, NB), 0)
    c0 = lax.broadcasted_iota(jnp.int32, (NB, NB), 1)
    lt = (r0 < c0).astype(_F32)
    starts = jnp.dot(counts, lt, preferred_element_type=_F32)

    tb = 512
    nblk = N // tb
    r1 = lax.broadcasted_iota(jnp.int32, (tb, tb), 0)
    c1 = lax.broadcasted_iota(jnp.int32, (tb, tb), 1)
    lstrict = (c1 < r1).astype(_F32)

    run = starts
    for i in range(nblk):
        ob = lax.slice(onehot, (i * tb, 0), ((i + 1) * tb, NB))
        cb = jnp.dot(lstrict, ob, preferred_element_type=_F32) + run
        pb = jnp.sum(cb * ob, axis=1, keepdims=True)
        run = run + jnp.sum(ob, axis=0, keepdims=True)
        g_ref[0, i, :] = pb[:, 0].astype(jnp.int32) + b * N


def _scatter_positions(hashes):
    h3 = hashes.reshape(D, N, 1)
    p = pl.pallas_call(
        _perm_body,
        grid=(D,),
        in_specs=[pl.BlockSpec((1, N, 1), lambda b: (b, 0, 0))],
        out_specs=pl.BlockSpec((1, N // 512, 512), lambda b: (b, 0, 0)),
        out_shape=jax.ShapeDtypeStruct((D, N // 512, 512), jnp.int32),
    )(h3)
    return p.reshape(T)


def _sc_permute(x2f, pos):
    info = plsc.get_sparse_core_info()
    nw = info.num_cores * info.num_subcores
    per_w = T // nw
    rows_chunk = 32
    nchunk = per_w // rows_chunk
    mesh = plsc.VectorSubcoreMesh(core_axis_name="c", subcore_axis_name="s")

    @functools.partial(
        pl.kernel,
        mesh=mesh,
        out_type=jax.ShapeDtypeStruct((T, H), _F32),
        scratch_types=[
            pltpu.VMEM((rows_chunk,), jnp.int32),
            pltpu.VMEM((rows_chunk, H), _F32),
            pltpu.SemaphoreType.DMA,
        ],
    )
    def scatter_k(tab_hbm, idx_hbm, out_hbm, idx_v, rows_v, sem):
        wid = lax.axis_index("s") * info.num_cores + lax.axis_index("c")
        base = wid * per_w
        for j in range(nchunk):
            off = base + j * rows_chunk
            pltpu.sync_copy(idx_hbm.at[pl.ds(off, rows_chunk)], idx_v)
            pltpu.sync_copy(tab_hbm.at[pl.ds(off, rows_chunk)], rows_v)
            pltpu.async_copy(rows_v, out_hbm.at[idx_v], sem).wait()

    return scatter_k(x2f, pos)


def _tail_body(x_ref, x2g_ref, wqk_ref, bqk_ref, wv_ref, bv_ref,
               wu_ref, bu_ref, n1g_ref, n1b_ref,
               w1_ref, b1_ref, w2_ref, b2_ref, n2g_ref, n2b_ref, o_ref):
    x2g = x2g_ref[...]
    qkf = jnp.dot(x2g, wqk_ref[...], preferred_element_type=_F32) + bqk_ref[...]
    valsf = jnp.dot(x2g, wv_ref[...], preferred_element_type=_F32) + bv_ref[...]
    rr = lax.broadcasted_iota(jnp.int32, (CH, CH), 0)
    cc = lax.broadcasted_iota(jnp.int32, (CH, CH), 1)
    parts = []
    for sub in range(BC):
        q = lax.slice(qkf, (sub * CH, 0), ((sub + 1) * CH, HD))
        v = lax.slice(valsf, (sub * CH, 0), ((sub + 1) * CH, HD))
        s = lax.dot_general(q, q, (((1,), (1,)), ((), ())),
                            preferred_element_type=_F32) * _F32(1.0 / 8.0)
        s = jnp.where(rr == cc, s / _F32(PEN), s)
        parts.append(jnp.dot(s, v, preferred_element_type=_F32))
    attn = jnp.concatenate(parts, axis=0)
    ua = jnp.dot(attn, wu_ref[...], preferred_element_type=_F32) + bu_ref[...]

    x1 = x_ref[:, :H]
    x2 = x_ref[:, H:]
    t1 = x1 + ua
    mu = jnp.mean(t1, axis=1, keepdims=True)
    var = jnp.mean((t1 - mu) ** 2, axis=1, keepdims=True)
    y1 = (t1 - mu) / jnp.sqrt(var + _F32(1e-5)) * n1g_ref[...] + n1b_ref[...]

    ff = jnp.maximum(jnp.dot(y1, w1_ref[...], preferred_element_type=_F32)
                     + b1_ref[...], _F32(0.0))
    ffo = jnp.dot(ff, w2_ref[...], preferred_element_type=_F32) + b2_ref[...]
    t2 = x2 + ffo
    mu2 = jnp.mean(t2, axis=1, keepdims=True)
    var2 = jnp.mean((t2 - mu2) ** 2, axis=1, keepdims=True)
    y2 = (t2 - mu2) / jnp.sqrt(var2 + _F32(1e-5)) * n2g_ref[...] + n2b_ref[...]

    o_ref[:, :H] = y1
    o_ref[:, H:] = y2


def _attention_tail(xf, x2g, wqk, bqk, wv, bv, wu, bu, n1g, n1b,
                    w1, b1, w2, b2, n2g, n2b):
    res = lambda shape: pl.BlockSpec(shape, lambda c: tuple(0 for _ in shape))
    return pl.pallas_call(
        _tail_body,
        grid=(NCHUNK // BC,),
        in_specs=[
            pl.BlockSpec((BC * CH, DMODEL), lambda c: (c, 0)),
            pl.BlockSpec((BC * CH, H), lambda c: (c, 0)),
            res((H, HD)), res((1, HD)),
            res((H, HD)), res((1, HD)),
            res((HD, H)), res((1, H)),
            res((1, H)), res((1, H)),
            res((H, FF)), res((1, FF)),
            res((FF, H)), res((1, H)),
            res((1, H)), res((1, H)),
        ],
        out_specs=pl.BlockSpec((BC * CH, DMODEL), lambda c: (c, 0)),
        out_shape=jax.ShapeDtypeStruct((T, DMODEL), _F32),
    )(xf, x2g, wqk, bqk, wv, bv, wu, bu, n1g, n1b, w1, b1, w2, b2, n2g, n2b)


def kernel(x, Wqk_w, Wqk_b, Wv_w, Wv_b, unify_w, unify_b, n1_g, n1_b,
           ff_w1, ff_b1, ff_w2, ff_b2, n2_g, n2_b, hashM):
    xf = x.reshape(T, DMODEL)
    r = lambda v: v.reshape(1, -1)
    x2f, hashes = _hashes_and_x2(xf, Wqk_w, r(Wqk_b), hashM)
    pos = _scatter_positions(hashes)
    x2g = _sc_permute(x2f, pos)
    out = _attention_tail(xf, x2g, Wqk_w, r(Wqk_b), Wv_w, r(Wv_b),
                          unify_w, r(unify_b), r(n1_g), r(n1_b),
                          ff_w1, r(ff_b1), ff_w2, r(ff_b2), r(n2_g), r(n2_b))
    return out.reshape(D, N, DMODEL)


# speedup vs baseline: 2.0583x; 1.1203x over previous
import functools

import jax
import jax.numpy as jnp
from jax import lax
from jax.experimental import pallas as pl
from jax.experimental.pallas import tpu as pltpu
from jax.experimental.pallas import tpu_sc as plsc

DMODEL = 1024
H = DMODEL // 2
DQK = 64
HEADS = 16
HD = HEADS * DQK
FF = 2048
NB = 64
PEN = 100000.0
D = 2
N = 4096
T = D * N
CH = 128
BC = 4
NCHUNK = T // CH

_F32 = jnp.float32


HBLK = 512
NHSTEP = T // HBLK


def _hash_sort_body(x_ref, wqk_ref, bqk_ref, hm_ref, x2_ref, p_ref, h_acc):
    i = pl.program_id(0)

    @pl.when(i < NHSTEP)
    def _hash():
        x2 = x_ref[...]
        x2_ref[...] = x2
        qk = jnp.dot(x2, wqk_ref[...],
                     preferred_element_type=_F32) + bqk_ref[...]
        proj = jnp.dot(qk, hm_ref[...], preferred_element_type=_F32)
        m = jnp.maximum(jnp.max(proj, axis=1, keepdims=True),
                        jnp.max(-proj, axis=1, keepdims=True))
        io = lax.broadcasted_iota(jnp.int32, (HBLK, NB // 2), 1)
        big = jnp.int32(2 * NB)
        i1 = jnp.min(jnp.where(proj == m, io, big), axis=1, keepdims=True)
        i2 = jnp.min(jnp.where(-proj == m, io + (NB // 2), big), axis=1,
                     keepdims=True)
        h_acc[pl.ds(i * HBLK, HBLK), :] = jnp.minimum(i1, i2)

    @pl.when(i >= NHSTEP)
    def _sort():
        b = i - NHSTEP
        h = h_acc[pl.ds(b * N, N), :]
        buckets = lax.broadcasted_iota(jnp.int32, (N, NB), 1)
        onehot = (h == buckets).astype(_F32)
        counts = jnp.sum(onehot, axis=0, keepdims=True)
        r0 = lax.broadcasted_iota(jnp.int32, (NB, NB), 0)
        c0 = lax.broadcasted_iota(jnp.int32, (NB, NB), 1)
        lt = (r0 < c0).astype(_F32)
        starts = jnp.dot(counts, lt, preferred_element_type=_F32)
        tb = 512
        nblk = N // tb
        r1 = lax.broadcasted_iota(jnp.int32, (tb, tb), 0)
        c1 = lax.broadcasted_iota(jnp.int32, (tb, tb), 1)
        lstrict = (c1 < r1).astype(_F32)
        run = starts
        for j in range(nblk):
            ob = lax.slice(onehot, (j * tb, 0), ((j + 1) * tb, NB))
            cb = jnp.dot(lstrict, ob, preferred_element_type=_F32) + run
            pb = jnp.sum(cb * ob, axis=1, keepdims=True)
            run = run + jnp.sum(ob, axis=0, keepdims=True)
            p_ref[0, j, :] = pb[:, 0].astype(jnp.int32) + b * N


def _hash_and_sort(xf, wqk, bqk, hm):
    nstep = NHSTEP + D
    x2f, p = pl.pallas_call(
        _hash_sort_body,
        grid=(nstep,),
        in_specs=[
            pl.BlockSpec((HBLK, H), lambda i: (jnp.minimum(i, NHSTEP - 1), 1)),
            pl.BlockSpec((H, HD), lambda i: (0, 0)),
            pl.BlockSpec((1, HD), lambda i: (0, 0)),
            pl.BlockSpec((HD, NB // 2), lambda i: (0, 0)),
        ],
        out_specs=[
            pl.BlockSpec((HBLK, H), lambda i: (jnp.minimum(i, NHSTEP - 1), 0)),
            pl.BlockSpec((1, N // 512, 512),
                         lambda i: (jnp.maximum(i - NHSTEP, 0), 0, 0)),
        ],
        out_shape=[
            jax.ShapeDtypeStruct((T, H), _F32),
            jax.ShapeDtypeStruct((D, N // 512, 512), jnp.int32),
        ],
        scratch_shapes=[pltpu.VMEM((T, 1), jnp.int32)],
        compiler_params=pltpu.CompilerParams(
            dimension_semantics=("arbitrary",)),
    )(xf, wqk, bqk, hm)
    return x2f, p.reshape(T)


def _sc_permute(x2f, pos):
    info = plsc.get_sparse_core_info()
    nw = info.num_cores * info.num_subcores
    per_w = T // nw
    rows_chunk = 32
    nchunk = per_w // rows_chunk
    mesh = plsc.VectorSubcoreMesh(core_axis_name="c", subcore_axis_name="s")

    @functools.partial(
        pl.kernel,
        mesh=mesh,
        out_type=jax.ShapeDtypeStruct((T, H), _F32),
        scratch_types=[
            pltpu.VMEM((rows_chunk,), jnp.int32),
            pltpu.VMEM((rows_chunk, H), _F32),
            pltpu.SemaphoreType.DMA,
        ],
    )
    def scatter_k(tab_hbm, idx_hbm, out_hbm, idx_v, rows_v, sem):
        wid = lax.axis_index("s") * info.num_cores + lax.axis_index("c")
        base = wid * per_w
        for j in range(nchunk):
            off = base + j * rows_chunk
            pltpu.sync_copy(idx_hbm.at[pl.ds(off, rows_chunk)], idx_v)
            pltpu.sync_copy(tab_hbm.at[pl.ds(off, rows_chunk)], rows_v)
            pltpu.async_copy(rows_v, out_hbm.at[idx_v], sem).wait()

    return scatter_k(x2f, pos)


def _tail_body(x_ref, x2g_ref, wqk_ref, bqk_ref, wv_ref, bv_ref,
               wu_ref, bu_ref, n1g_ref, n1b_ref,
               w1_ref, b1_ref, w2_ref, b2_ref, n2g_ref, n2b_ref, o_ref):
    x2g = x2g_ref[...]
    qkf = jnp.dot(x2g, wqk_ref[...], preferred_element_type=_F32) + bqk_ref[...]
    valsf = jnp.dot(x2g, wv_ref[...], preferred_element_type=_F32) + bv_ref[...]
    rr = lax.broadcasted_iota(jnp.int32, (CH, CH), 0)
    cc = lax.broadcasted_iota(jnp.int32, (CH, CH), 1)
    parts = []
    for sub in range(BC):
        q = lax.slice(qkf, (sub * CH, 0), ((sub + 1) * CH, HD))
        v = lax.slice(valsf, (sub * CH, 0), ((sub + 1) * CH, HD))
        s = lax.dot_general(q, q, (((1,), (1,)), ((), ())),
                            preferred_element_type=_F32) * _F32(1.0 / 8.0)
        s = jnp.where(rr == cc, s / _F32(PEN), s)
        parts.append(jnp.dot(s, v, preferred_element_type=_F32))
    attn = jnp.concatenate(parts, axis=0)
    ua = jnp.dot(attn, wu_ref[...], preferred_element_type=_F32) + bu_ref[...]

    x1 = x_ref[:, :H]
    x2 = x_ref[:, H:]
    t1 = x1 + ua
    mu = jnp.mean(t1, axis=1, keepdims=True)
    var = jnp.mean((t1 - mu) ** 2, axis=1, keepdims=True)
    y1 = (t1 - mu) / jnp.sqrt(var + _F32(1e-5)) * n1g_ref[...] + n1b_ref[...]

    ff = jnp.maximum(jnp.dot(y1, w1_ref[...], preferred_element_type=_F32)
                     + b1_ref[...], _F32(0.0))
    ffo = jnp.dot(ff, w2_ref[...], preferred_element_type=_F32) + b2_ref[...]
    t2 = x2 + ffo
    mu2 = jnp.mean(t2, axis=1, keepdims=True)
    var2 = jnp.mean((t2 - mu2) ** 2, axis=1, keepdims=True)
    y2 = (t2 - mu2) / jnp.sqrt(var2 + _F32(1e-5)) * n2g_ref[...] + n2b_ref[...]

    o_ref[:, :H] = y1
    o_ref[:, H:] = y2


def _attention_tail(xf, x2g, wqk, bqk, wv, bv, wu, bu, n1g, n1b,
                    w1, b1, w2, b2, n2g, n2b):
    res = lambda shape: pl.BlockSpec(shape, lambda c: tuple(0 for _ in shape))
    return pl.pallas_call(
        _tail_body,
        grid=(NCHUNK // BC,),
        in_specs=[
            pl.BlockSpec((BC * CH, DMODEL), lambda c: (c, 0)),
            pl.BlockSpec((BC * CH, H), lambda c: (c, 0)),
            res((H, HD)), res((1, HD)),
            res((H, HD)), res((1, HD)),
            res((HD, H)), res((1, H)),
            res((1, H)), res((1, H)),
            res((H, FF)), res((1, FF)),
            res((FF, H)), res((1, H)),
            res((1, H)), res((1, H)),
        ],
        out_specs=pl.BlockSpec((BC * CH, DMODEL), lambda c: (c, 0)),
        out_shape=jax.ShapeDtypeStruct((T, DMODEL), _F32),
    )(xf, x2g, wqk, bqk, wv, bv, wu, bu, n1g, n1b, w1, b1, w2, b2, n2g, n2b)


def kernel(x, Wqk_w, Wqk_b, Wv_w, Wv_b, unify_w, unify_b, n1_g, n1_b,
           ff_w1, ff_b1, ff_w2, ff_b2, n2_g, n2_b, hashM):
    xf = x.reshape(T, DMODEL)
    r = lambda v: v.reshape(1, -1)
    x2f, pos = _hash_and_sort(xf, Wqk_w, r(Wqk_b), hashM)
    x2g = _sc_permute(x2f, pos)
    out = _attention_tail(xf, x2g, Wqk_w, r(Wqk_b), Wv_w, r(Wv_b),
                          unify_w, r(unify_b), r(n1_g), r(n1_b),
                          ff_w1, r(ff_b1), ff_w2, r(ff_b2), r(n2_g), r(n2_b))
    return out.reshape(D, N, DMODEL)
